# Initial kernel scaffold; baseline (speedup 1.0000x reference)
import jax
import jax.numpy as jnp
from jax.experimental import pallas as pl


def kernel(sentence, table):
    raise NotImplementedError("write your pallas kernel here")


# speedup vs baseline: 2.2613x; 2.2613x over previous
import functools

import jax
import jax.numpy as jnp
from jax import lax
from jax.experimental import pallas as pl
from jax.experimental.pallas import tpu as pltpu
from jax.experimental.pallas import tpu_sc as plsc

CHUNK = 128
K = 10


@functools.cache
def _build(N, V, D):
    NW = 32
    per_w = N // NW
    n_chunks = per_w // CHUNK
    n_groups = n_chunks // K
    assert per_w * NW == N and n_chunks * CHUNK == per_w
    assert n_groups * K == n_chunks

    mesh = plsc.VectorSubcoreMesh(core_axis_name="c", subcore_axis_name="s")

    @functools.partial(
        pl.kernel,
        mesh=mesh,
        compiler_params=pltpu.CompilerParams(use_tc_tiling_on_sc=False),
        out_type=jax.ShapeDtypeStruct((N, D), jnp.float32),
        scratch_types=[
            pltpu.VMEM((n_chunks, CHUNK), jnp.int32),
            pltpu.VMEM((K * CHUNK, D), jnp.float32),
            pltpu.SemaphoreType.DMA,
        ],
    )
    def emb(idx_hbm, table_hbm, out_hbm, idx_v, rows_v, gsem):
        wid = lax.axis_index("s") * 2 + lax.axis_index("c")
        base_chunk = wid * n_chunks
        out_base = wid * per_w

        pltpu.sync_copy(idx_hbm.at[pl.ds(base_chunk, n_chunks)], idx_v)

        def group(g, _):
            for b in range(K):
                pltpu.make_async_copy(
                    table_hbm.at[idx_v.at[g * K + b]],
                    rows_v.at[pl.ds(b * CHUNK, CHUNK)],
                    gsem,
                ).start()
            pltpu.make_async_copy(
                table_hbm.at[pl.ds(0, K * CHUNK)], rows_v, gsem
            ).wait()
            pltpu.sync_copy(
                rows_v, out_hbm.at[pl.ds(out_base + g * (K * CHUNK), K * CHUNK)]
            )
            return 0

        lax.fori_loop(0, n_groups, group, 0)

    return emb


def kernel(sentence, table):
    B, S = sentence.shape
    V, D = table.shape
    N = B * S
    idx = sentence.reshape(N // CHUNK, CHUNK).astype(jnp.int32)
    out = _build(N, V, D)(idx, table)
    return out.reshape(B, S * D)


# speedup vs baseline: 2.2715x; 1.0045x over previous
import functools

import jax
import jax.numpy as jnp
from jax import lax
from jax.experimental import pallas as pl
from jax.experimental.pallas import tpu as pltpu
from jax.experimental.pallas import tpu_sc as plsc

G = 8


@functools.cache
def _build(B, S, V, D):
    NW = 32
    rows_w = B // NW
    n_groups = rows_w // G
    assert rows_w * NW == B and n_groups * G == rows_w
    splits = []
    off = 0
    while off < S:
        n = min(128, S - off)
        splits.append((off, n))
        off += n
    assert all(o % 8 == 0 for o, _ in splits)

    mesh = plsc.VectorSubcoreMesh(core_axis_name="c", subcore_axis_name="s")

    @functools.partial(
        pl.kernel,
        mesh=mesh,
        compiler_params=pltpu.CompilerParams(use_tc_tiling_on_sc=False),
        out_type=jax.ShapeDtypeStruct((B * S, D), jnp.float32),
        scratch_types=[
            pltpu.VMEM((rows_w, S), jnp.int32),
            pltpu.VMEM((G * S, D), jnp.float32),
            pltpu.SemaphoreType.DMA,
        ],
    )
    def emb(idx_hbm, table_hbm, out_hbm, idx_v, rows_v, gsem):
        wid = lax.axis_index("s") * 2 + lax.axis_index("c")
        row0 = wid * rows_w

        pltpu.sync_copy(idx_hbm.at[pl.ds(row0, rows_w)], idx_v)

        def group(g, _):
            for r in range(G):
                for off, n in splits:
                    pltpu.make_async_copy(
                        table_hbm.at[idx_v.at[g * G + r, pl.ds(off, n)]],
                        rows_v.at[pl.ds(r * S + off, n)],
                        gsem,
                    ).start()
            pltpu.make_async_copy(
                table_hbm.at[pl.ds(0, G * S)], rows_v, gsem
            ).wait()
            pltpu.sync_copy(
                rows_v, out_hbm.at[pl.ds((row0 + g * G) * S, G * S)]
            )
            return 0

        lax.fori_loop(0, n_groups, group, 0)

    return emb


def kernel(sentence, table):
    B, S = sentence.shape
    V, D = table.shape
    out = _build(B, S, V, D)(sentence.astype(jnp.int32), table)
    return out.reshape(B, S * D)
